# baseline (speedup 1.0000x reference)
import jax
import jax.numpy as jnp
from jax.experimental import pallas as pl
from jax.experimental.pallas import tpu as pltpu
from jax.experimental.pallas import tpu_sc as plsc

_D = 768
_FF = 256
_E = 64
_K = 2
_T = 2048
_FFS = 512
_BT = 64
_MT = 128
_NSLOT = _MT * _BT
_NA = _K * _T
_CH = 256
_SC_NC = 2
_SC_NS = 16
_NW = _SC_NC * _SC_NS


def _router_body(x_ref, wg_ref, w01_ref, pos_ref, texp_ref, tslot_ref,
                 nact_ref, ef_scr, rank_scr):
    x = x_ref[...]
    wg = wg_ref[...]
    logits = jax.lax.dot_general(
        x, wg, (((1,), (1,)), ((), ())),
        preferred_element_type=jnp.float32)
    colid = jax.lax.broadcasted_iota(jnp.int32, (_T, _E), 1)
    m1 = jnp.max(logits, axis=1, keepdims=True)
    i1 = jnp.min(jnp.where(logits == m1, colid, _E), axis=1, keepdims=True)
    lmask = jnp.where(colid == i1, -jnp.inf, logits)
    m2 = jnp.max(lmask, axis=1, keepdims=True)
    i2 = jnp.min(jnp.where(lmask == m2, colid, _E), axis=1, keepdims=True)
    s1 = jax.nn.sigmoid(m1)
    s2 = jax.nn.sigmoid(m2)
    den = s1 + s2 + 1e-10
    w01_ref[:, 0:1] = s1 / den
    w01_ref[:, 1:2] = s2 / den
    ef_scr[0:_T, :] = i1
    ef_scr[_T:_NA, :] = i2

    tri = jnp.where(
        jax.lax.broadcasted_iota(jnp.int32, (_CH, _CH), 0)
        > jax.lax.broadcasted_iota(jnp.int32, (_CH, _CH), 1),
        1.0, 0.0).astype(jnp.float32)
    eidrow = jax.lax.broadcasted_iota(jnp.int32, (_CH, _E), 1)
    nch = _NA // _CH

    def pass1(ci, carry):
        ec = ef_scr[pl.ds(ci * _CH, _CH), :]
        oh = jnp.where(ec == eidrow, 1.0, 0.0).astype(jnp.float32)
        cume = jax.lax.dot_general(
            tri, oh, (((1,), (0,)), ((), ())),
            preferred_element_type=jnp.float32,
            precision=jax.lax.Precision.HIGHEST)
        rank = jnp.sum((cume + carry) * oh, axis=1, keepdims=True)
        rank_scr[pl.ds(ci * _CH, _CH), :] = rank
        return carry + jnp.sum(oh, axis=0, keepdims=True)

    counts = jax.lax.fori_loop(0, nch, pass1, jnp.zeros((1, _E), jnp.float32))
    ci_ = counts.astype(jnp.int32)
    pg = ((ci_ + (_BT - 1)) // _BT) * _BT
    pgf = pg.astype(jnp.float32)
    upp = jnp.where(
        jax.lax.broadcasted_iota(jnp.int32, (_E, _E), 0)
        < jax.lax.broadcasted_iota(jnp.int32, (_E, _E), 1),
        1.0, 0.0).astype(jnp.float32)
    poff = jax.lax.dot_general(
        pgf, upp, (((1,), (0,)), ((), ())),
        preferred_element_type=jnp.float32,
        precision=jax.lax.Precision.HIGHEST)
    poff_next = poff + pgf

    def pass2(ci, _):
        ec = ef_scr[pl.ds(ci * _CH, _CH), :]
        oh = jnp.where(ec == eidrow, 1.0, 0.0).astype(jnp.float32)
        posc = rank_scr[pl.ds(ci * _CH, _CH), :] + jnp.sum(
            oh * poff, axis=1, keepdims=True)
        pos_ref[pl.ds(ci * _CH, _CH), :] = posc.astype(jnp.int32)
        return 0

    jax.lax.fori_loop(0, nch, pass2, 0)

    nact = jnp.sum(pg) // _BT
    midc = jax.lax.broadcasted_iota(jnp.int32, (_MT, 1), 0)
    rbase = (midc * _BT).astype(jnp.float32)
    seg_end_le = jnp.where(poff_next <= rbase, 1.0, 0.0)
    texp_raw = jnp.minimum(
        jnp.sum(seg_end_le, axis=1, keepdims=True).astype(jnp.int32), _E - 1)
    texp_last = jnp.sum(jnp.where(midc == nact - 1, texp_raw, 0))
    texp_ref[...] = jnp.where(midc < nact, texp_raw, texp_last)
    tslot_ref[...] = jnp.where(midc < nact, midc, nact - 1)
    nact_ref[...] = jnp.reshape(nact, (1, 1))


def _run_router(x2d, wg):
    return pl.pallas_call(
        _router_body,
        out_shape=[
            jax.ShapeDtypeStruct((_T, _K), jnp.float32),
            jax.ShapeDtypeStruct((_NA, 1), jnp.int32),
            jax.ShapeDtypeStruct((_MT, 1), jnp.int32),
            jax.ShapeDtypeStruct((_MT, 1), jnp.int32),
            jax.ShapeDtypeStruct((1, 1), jnp.int32),
        ],
        scratch_shapes=[
            pltpu.VMEM((_NA, 1), jnp.int32),
            pltpu.VMEM((_NA, 1), jnp.float32),
        ],
    )(x2d, wg)


def _dispatch_body(x_hbm, pos_hbm, xs_hbm, rows_v, idx0_v, idx1_v, sem):
    c = jax.lax.axis_index("c")
    s = jax.lax.axis_index("s")
    wid = s * _SC_NC + c
    ntok = _T // _NW
    t0 = wid * ntok
    pltpu.sync_copy(pos_hbm.at[pl.ds(t0, ntok)], idx0_v)
    pltpu.sync_copy(pos_hbm.at[pl.ds(_T + t0, ntok)], idx1_v)
    pltpu.sync_copy(x_hbm.at[pl.ds(t0, ntok)], rows_v)
    cp0 = pltpu.make_async_copy(rows_v, xs_hbm.at[idx0_v], sem)
    cp1 = pltpu.make_async_copy(rows_v, xs_hbm.at[idx1_v], sem)
    cp0.start()
    cp1.start()
    cp0.wait()
    cp1.wait()


def _run_dispatch(x2d, pos_flat):
    return pl.kernel(
        _dispatch_body,
        out_type=jax.ShapeDtypeStruct((_NSLOT, _D), jnp.float32),
        mesh=plsc.VectorSubcoreMesh(core_axis_name="c", subcore_axis_name="s"),
        scratch_types=[
            pltpu.VMEM((_T // _NW, _D), jnp.float32),
            pltpu.VMEM((_T // _NW,), jnp.int32),
            pltpu.VMEM((_T // _NW,), jnp.int32),
            pltpu.SemaphoreType.DMA,
        ],
    )(x2d, pos_flat)


def _gemm_body(te_ref, ts_ref, na_ref, xs_ref, w1_ref, b1_ref, w3_ref,
               b3_ref, w2_ref, b2_ref, z_ref):
    m = pl.program_id(0)

    @pl.when(m < na_ref[0])
    def _():
        xt = xs_ref[...].astype(jnp.bfloat16)

        def dotc(a, b):
            return jax.lax.dot_general(
                a, b.astype(jnp.bfloat16), (((1,), (1,)), ((), ())),
                preferred_element_type=jnp.float32)

        h1 = dotc(xt, w1_ref[0]) + b1_ref[0]
        h3 = dotc(xt, w3_ref[0]) + b3_ref[0]
        h = (h1 * jax.nn.sigmoid(h1) * h3).astype(jnp.bfloat16)
        z_ref[...] = dotc(h, w2_ref[0]) + b2_ref[0]


def _run_gemm(texp, tslot, nact, xs, w1, b1, w3, b3, w2, b2):
    grid_spec = pltpu.PrefetchScalarGridSpec(
        num_scalar_prefetch=3,
        grid=(_MT,),
        in_specs=[
            pl.BlockSpec((_BT, _D), lambda m, te, ts, na: (ts[m], 0)),
            pl.BlockSpec((1, _FF, _D), lambda m, te, ts, na: (te[m], 0, 0)),
            pl.BlockSpec((1, 1, _FF), lambda m, te, ts, na: (te[m], 0, 0)),
            pl.BlockSpec((1, _FF, _D), lambda m, te, ts, na: (te[m], 0, 0)),
            pl.BlockSpec((1, 1, _FF), lambda m, te, ts, na: (te[m], 0, 0)),
            pl.BlockSpec((1, _D, _FF), lambda m, te, ts, na: (te[m], 0, 0)),
            pl.BlockSpec((1, 1, _D), lambda m, te, ts, na: (te[m], 0, 0)),
        ],
        out_specs=pl.BlockSpec((_BT, _D), lambda m, te, ts, na: (ts[m], 0)),
    )
    return pl.pallas_call(
        _gemm_body,
        grid_spec=grid_spec,
        out_shape=jax.ShapeDtypeStruct((_NSLOT, _D), jnp.float32),
    )(texp, tslot, nact, xs, w1, b1, w3, b3, w2, b2)


def _gatherz_body(z_hbm, pos_hbm, zg_hbm, rows_v, idx_v, sem):
    c = jax.lax.axis_index("c")
    s = jax.lax.axis_index("s")
    wid = s * _SC_NC + c
    nrow = _NA // _NW
    r0 = wid * nrow
    pltpu.sync_copy(pos_hbm.at[pl.ds(r0, nrow)], idx_v)
    pltpu.async_copy(z_hbm.at[idx_v], rows_v, sem).wait()
    pltpu.sync_copy(rows_v, zg_hbm.at[pl.ds(r0, nrow)])


def _run_gatherz(z, pos_flat):
    return pl.kernel(
        _gatherz_body,
        out_type=jax.ShapeDtypeStruct((_NA, _D), jnp.float32),
        mesh=plsc.VectorSubcoreMesh(core_axis_name="c", subcore_axis_name="s"),
        scratch_types=[
            pltpu.VMEM((_NA // _NW, _D), jnp.float32),
            pltpu.VMEM((_NA // _NW,), jnp.int32),
            pltpu.SemaphoreType.DMA,
        ],
    )(z, pos_flat)


_TB = 128


def _shared_body(x_ref, sw1_ref, sb1_ref, sw3_ref, sb3_ref, sw2_ref,
                 sb2_ref, sh_ref):
    x = x_ref[...]
    g1 = jax.lax.dot_general(
        x, sw1_ref[...], (((1,), (1,)), ((), ())),
        preferred_element_type=jnp.float32) + sb1_ref[...]
    g3 = jax.lax.dot_general(
        x, sw3_ref[...], (((1,), (1,)), ((), ())),
        preferred_element_type=jnp.float32) + sb3_ref[...]
    hs = g1 * jax.nn.sigmoid(g1) * g3
    sh_ref[...] = jax.lax.dot_general(
        hs, sw2_ref[...], (((1,), (1,)), ((), ())),
        preferred_element_type=jnp.float32) + sb2_ref[...]


def _run_shared(x2d, sw1, sb1, sw3, sb3, sw2, sb2):
    nb = _T // _TB
    return pl.pallas_call(
        _shared_body,
        grid=(nb,),
        in_specs=[
            pl.BlockSpec((_TB, _D), lambda i: (i, 0)),
            pl.BlockSpec((_FFS, _D), lambda i: (0, 0)),
            pl.BlockSpec((1, _FFS), lambda i: (0, 0)),
            pl.BlockSpec((_FFS, _D), lambda i: (0, 0)),
            pl.BlockSpec((1, _FFS), lambda i: (0, 0)),
            pl.BlockSpec((_D, _FFS), lambda i: (0, 0)),
            pl.BlockSpec((1, _D), lambda i: (0, 0)),
        ],
        out_specs=pl.BlockSpec((_TB, _D), lambda i: (i, 0)),
        out_shape=jax.ShapeDtypeStruct((_T, _D), jnp.float32),
    )(x2d, sw1, sb1, sw3, sb3, sw2, sb2)


def _combine_body(zg0_ref, zg1_ref, w_ref, sh_ref, out_ref):
    out_ref[...] = (w_ref[:, 0:1] * zg0_ref[...]
                    + w_ref[:, 1:2] * zg1_ref[...] + sh_ref[...])


def _run_combine(zg, w01, sh):
    nb = _T // _TB
    return pl.pallas_call(
        _combine_body,
        grid=(nb,),
        in_specs=[
            pl.BlockSpec((_TB, _D), lambda i: (i, 0)),
            pl.BlockSpec((_TB, _D), lambda i: (i + nb, 0)),
            pl.BlockSpec((_TB, _K), lambda i: (i, 0)),
            pl.BlockSpec((_TB, _D), lambda i: (i, 0)),
        ],
        out_specs=pl.BlockSpec((_TB, _D), lambda i: (i, 0)),
        out_shape=jax.ShapeDtypeStruct((_T, _D), jnp.float32),
    )(zg, zg, w01, sh)


def kernel(x, Wg, W1, b1, W3, b3, W2, b2, sw1, sb1, sw3, sb3, sw2, sb2):
    bsz, seq_len, d = x.shape
    x2d = x.reshape(-1, d)

    w01, pos2d, texp2d, tslot2d, nact2d = _run_router(x2d, Wg)
    pos_flat = pos2d.reshape(_NA)
    texp = texp2d.reshape(_MT)
    tslot = tslot2d.reshape(_MT)
    nact = nact2d.reshape(1)

    xs = _run_dispatch(x2d, pos_flat)
    z = _run_gemm(texp, tslot, nact, xs, W1,
                  b1.reshape(_E, 1, _FF), W3, b3.reshape(_E, 1, _FF),
                  W2, b2.reshape(_E, 1, _D))
    zg = _run_gatherz(z, pos_flat)
    sh = _run_shared(x2d, sw1, sb1.reshape(1, _FFS),
                     sw3, sb3.reshape(1, _FFS), sw2, sb2.reshape(1, _D))
    out2d = _run_combine(zg, w01, sh)
    return out2d.reshape(bsz, seq_len, d)


# speedup vs baseline: 1.0303x; 1.0303x over previous
import jax
import jax.numpy as jnp
from jax.experimental import pallas as pl
from jax.experimental.pallas import tpu as pltpu
from jax.experimental.pallas import tpu_sc as plsc

_D = 768
_FF = 256
_E = 64
_K = 2
_T = 2048
_FFS = 512
_BT = 64
_MT = 128
_NSLOT = _MT * _BT
_NA = _K * _T
_CH = 256
_SC_NC = 2
_SC_NS = 16
_NW = _SC_NC * _SC_NS


def _router_body(x_ref, wg_ref, w01_ref, pos_ref, texp_ref, tslot_ref,
                 nact_ref, ef_scr, rank_scr):
    x = x_ref[...]
    wg = wg_ref[...]
    logits = jax.lax.dot_general(
        x, wg, (((1,), (1,)), ((), ())),
        preferred_element_type=jnp.float32)
    colid = jax.lax.broadcasted_iota(jnp.int32, (_T, _E), 1)
    m1 = jnp.max(logits, axis=1, keepdims=True)
    i1 = jnp.min(jnp.where(logits == m1, colid, _E), axis=1, keepdims=True)
    lmask = jnp.where(colid == i1, -jnp.inf, logits)
    m2 = jnp.max(lmask, axis=1, keepdims=True)
    i2 = jnp.min(jnp.where(lmask == m2, colid, _E), axis=1, keepdims=True)
    s1 = jax.nn.sigmoid(m1)
    s2 = jax.nn.sigmoid(m2)
    den = s1 + s2 + 1e-10
    w01_ref[:, 0:1] = s1 / den
    w01_ref[:, 1:2] = s2 / den
    ef_scr[0:_T, :] = i1
    ef_scr[_T:_NA, :] = i2

    tri = jnp.where(
        jax.lax.broadcasted_iota(jnp.int32, (_CH, _CH), 0)
        > jax.lax.broadcasted_iota(jnp.int32, (_CH, _CH), 1),
        1.0, 0.0).astype(jnp.float32)
    eidrow = jax.lax.broadcasted_iota(jnp.int32, (_CH, _E), 1)
    nch = _NA // _CH

    def pass1(ci, carry):
        ec = ef_scr[pl.ds(ci * _CH, _CH), :]
        oh = jnp.where(ec == eidrow, 1.0, 0.0).astype(jnp.float32)
        cume = jax.lax.dot_general(
            tri, oh, (((1,), (0,)), ((), ())),
            preferred_element_type=jnp.float32,
            precision=jax.lax.Precision.HIGHEST)
        rank = jnp.sum((cume + carry) * oh, axis=1, keepdims=True)
        rank_scr[pl.ds(ci * _CH, _CH), :] = rank
        return carry + jnp.sum(oh, axis=0, keepdims=True)

    counts = jax.lax.fori_loop(0, nch, pass1, jnp.zeros((1, _E), jnp.float32))
    ci_ = counts.astype(jnp.int32)
    pg = ((ci_ + (_BT - 1)) // _BT) * _BT
    pgf = pg.astype(jnp.float32)
    upp = jnp.where(
        jax.lax.broadcasted_iota(jnp.int32, (_E, _E), 0)
        < jax.lax.broadcasted_iota(jnp.int32, (_E, _E), 1),
        1.0, 0.0).astype(jnp.float32)
    poff = jax.lax.dot_general(
        pgf, upp, (((1,), (0,)), ((), ())),
        preferred_element_type=jnp.float32,
        precision=jax.lax.Precision.HIGHEST)
    poff_next = poff + pgf

    def pass2(ci, _):
        ec = ef_scr[pl.ds(ci * _CH, _CH), :]
        oh = jnp.where(ec == eidrow, 1.0, 0.0).astype(jnp.float32)
        posc = rank_scr[pl.ds(ci * _CH, _CH), :] + jnp.sum(
            oh * poff, axis=1, keepdims=True)
        pos_ref[pl.ds(ci * _CH, _CH), :] = posc.astype(jnp.int32)
        return 0

    jax.lax.fori_loop(0, nch, pass2, 0)

    nact = jnp.sum(pg) // _BT
    midc = jax.lax.broadcasted_iota(jnp.int32, (_MT, 1), 0)
    rbase = (midc * _BT).astype(jnp.float32)
    seg_end_le = jnp.where(poff_next <= rbase, 1.0, 0.0)
    texp_raw = jnp.minimum(
        jnp.sum(seg_end_le, axis=1, keepdims=True).astype(jnp.int32), _E - 1)
    texp_last = jnp.sum(jnp.where(midc == nact - 1, texp_raw, 0))
    texp_ref[...] = jnp.where(midc < nact, texp_raw, texp_last)
    tslot_ref[...] = jnp.where(midc < nact, midc, nact - 1)
    nact_ref[...] = jnp.reshape(nact, (1, 1))


def _run_router(x2d, wg):
    return pl.pallas_call(
        _router_body,
        out_shape=[
            jax.ShapeDtypeStruct((_T, _K), jnp.float32),
            jax.ShapeDtypeStruct((_NA, 1), jnp.int32),
            jax.ShapeDtypeStruct((_MT, 1), jnp.int32),
            jax.ShapeDtypeStruct((_MT, 1), jnp.int32),
            jax.ShapeDtypeStruct((1, 1), jnp.int32),
        ],
        scratch_shapes=[
            pltpu.VMEM((_NA, 1), jnp.int32),
            pltpu.VMEM((_NA, 1), jnp.float32),
        ],
    )(x2d, wg)


def _dispatch_body(x_hbm, pos_hbm, xs_hbm, rows_v, idx0_v, idx1_v, sem):
    c = jax.lax.axis_index("c")
    s = jax.lax.axis_index("s")
    wid = s * _SC_NC + c
    ntok = _T // _NW
    t0 = wid * ntok
    pltpu.sync_copy(pos_hbm.at[pl.ds(t0, ntok)], idx0_v)
    pltpu.sync_copy(pos_hbm.at[pl.ds(_T + t0, ntok)], idx1_v)
    pltpu.sync_copy(x_hbm.at[pl.ds(t0, ntok)], rows_v)
    cp0 = pltpu.make_async_copy(rows_v, xs_hbm.at[idx0_v], sem)
    cp1 = pltpu.make_async_copy(rows_v, xs_hbm.at[idx1_v], sem)
    cp0.start()
    cp1.start()
    cp0.wait()
    cp1.wait()


def _run_dispatch(x2d, pos_flat):
    return pl.kernel(
        _dispatch_body,
        out_type=jax.ShapeDtypeStruct((_NSLOT, _D), jnp.float32),
        mesh=plsc.VectorSubcoreMesh(core_axis_name="c", subcore_axis_name="s"),
        scratch_types=[
            pltpu.VMEM((_T // _NW, _D), jnp.float32),
            pltpu.VMEM((_T // _NW,), jnp.int32),
            pltpu.VMEM((_T // _NW,), jnp.int32),
            pltpu.SemaphoreType.DMA,
        ],
    )(x2d, pos_flat)


def _gemm_body(te_ref, ts_ref, na_ref, xs_ref, w1_ref, b1_ref, w3_ref,
               b3_ref, w2_ref, b2_ref, z_ref):
    m = pl.program_id(0)

    @pl.when(m < na_ref[0])
    def _():
        xt = xs_ref[...].astype(jnp.bfloat16)

        def dotc(a, b):
            return jax.lax.dot_general(
                a, b.astype(jnp.bfloat16), (((1,), (1,)), ((), ())),
                preferred_element_type=jnp.float32)

        h1 = dotc(xt, w1_ref[0]) + b1_ref[0]
        h3 = dotc(xt, w3_ref[0]) + b3_ref[0]
        h = (h1 * jax.nn.sigmoid(h1) * h3).astype(jnp.bfloat16)
        z_ref[...] = dotc(h, w2_ref[0]) + b2_ref[0]


def _run_gemm(texp, tslot, nact, xs, w1, b1, w3, b3, w2, b2):
    grid_spec = pltpu.PrefetchScalarGridSpec(
        num_scalar_prefetch=3,
        grid=(_MT,),
        in_specs=[
            pl.BlockSpec((_BT, _D), lambda m, te, ts, na: (ts[m], 0)),
            pl.BlockSpec((1, _FF, _D), lambda m, te, ts, na: (te[m], 0, 0)),
            pl.BlockSpec((1, 1, _FF), lambda m, te, ts, na: (te[m], 0, 0)),
            pl.BlockSpec((1, _FF, _D), lambda m, te, ts, na: (te[m], 0, 0)),
            pl.BlockSpec((1, 1, _FF), lambda m, te, ts, na: (te[m], 0, 0)),
            pl.BlockSpec((1, _D, _FF), lambda m, te, ts, na: (te[m], 0, 0)),
            pl.BlockSpec((1, 1, _D), lambda m, te, ts, na: (te[m], 0, 0)),
        ],
        out_specs=pl.BlockSpec((_BT, _D), lambda m, te, ts, na: (ts[m], 0)),
    )
    return pl.pallas_call(
        _gemm_body,
        grid_spec=grid_spec,
        out_shape=jax.ShapeDtypeStruct((_NSLOT, _D), jnp.float32),
    )(texp, tslot, nact, xs, w1, b1, w3, b3, w2, b2)


def _gatherz_body(z_hbm, pos_hbm, zg_hbm, rows_v, idx_v, sem):
    c = jax.lax.axis_index("c")
    s = jax.lax.axis_index("s")
    wid = s * _SC_NC + c
    nrow = _NA // _NW
    r0 = wid * nrow
    pltpu.sync_copy(pos_hbm.at[pl.ds(r0, nrow)], idx_v)
    pltpu.async_copy(z_hbm.at[idx_v], rows_v, sem).wait()
    pltpu.sync_copy(rows_v, zg_hbm.at[pl.ds(r0, nrow)])


def _run_gatherz(z, pos_flat):
    return pl.kernel(
        _gatherz_body,
        out_type=jax.ShapeDtypeStruct((_NA, _D), jnp.float32),
        mesh=plsc.VectorSubcoreMesh(core_axis_name="c", subcore_axis_name="s"),
        scratch_types=[
            pltpu.VMEM((_NA // _NW, _D), jnp.float32),
            pltpu.VMEM((_NA // _NW,), jnp.int32),
            pltpu.SemaphoreType.DMA,
        ],
    )(z, pos_flat)


_TB = 128


def _combine_body(x_ref, zg0_ref, zg1_ref, w_ref, sw1_ref, sb1_ref, sw3_ref,
                  sb3_ref, sw2_ref, sb2_ref, out_ref):
    x = x_ref[...]
    g1 = jax.lax.dot_general(
        x, sw1_ref[...], (((1,), (1,)), ((), ())),
        preferred_element_type=jnp.float32) + sb1_ref[...]
    g3 = jax.lax.dot_general(
        x, sw3_ref[...], (((1,), (1,)), ((), ())),
        preferred_element_type=jnp.float32) + sb3_ref[...]
    hs = g1 * jax.nn.sigmoid(g1) * g3
    sh = jax.lax.dot_general(
        hs, sw2_ref[...], (((1,), (1,)), ((), ())),
        preferred_element_type=jnp.float32) + sb2_ref[...]
    out_ref[...] = (w_ref[:, 0:1] * zg0_ref[...]
                    + w_ref[:, 1:2] * zg1_ref[...] + sh)


def _run_combine(x2d, zg, w01, sw1, sb1, sw3, sb3, sw2, sb2):
    nb = _T // _TB
    return pl.pallas_call(
        _combine_body,
        grid=(nb,),
        in_specs=[
            pl.BlockSpec((_TB, _D), lambda i: (i, 0)),
            pl.BlockSpec((_TB, _D), lambda i: (i, 0)),
            pl.BlockSpec((_TB, _D), lambda i: (i + nb, 0)),
            pl.BlockSpec((_TB, _K), lambda i: (i, 0)),
            pl.BlockSpec((_FFS, _D), lambda i: (0, 0)),
            pl.BlockSpec((1, _FFS), lambda i: (0, 0)),
            pl.BlockSpec((_FFS, _D), lambda i: (0, 0)),
            pl.BlockSpec((1, _FFS), lambda i: (0, 0)),
            pl.BlockSpec((_D, _FFS), lambda i: (0, 0)),
            pl.BlockSpec((1, _D), lambda i: (0, 0)),
        ],
        out_specs=pl.BlockSpec((_TB, _D), lambda i: (i, 0)),
        out_shape=jax.ShapeDtypeStruct((_T, _D), jnp.float32),
    )(x2d, zg, zg, w01, sw1, sb1, sw3, sb3, sw2, sb2)


def kernel(x, Wg, W1, b1, W3, b3, W2, b2, sw1, sb1, sw3, sb3, sw2, sb2):
    bsz, seq_len, d = x.shape
    x2d = x.reshape(-1, d)

    w01, pos2d, texp2d, tslot2d, nact2d = _run_router(x2d, Wg)
    pos_flat = pos2d.reshape(_NA)
    texp = texp2d.reshape(_MT)
    tslot = tslot2d.reshape(_MT)
    nact = nact2d.reshape(1)

    xs = _run_dispatch(x2d, pos_flat)
    z = _run_gemm(texp, tslot, nact, xs, W1,
                  b1.reshape(_E, 1, _FF), W3, b3.reshape(_E, 1, _FF),
                  W2, b2.reshape(_E, 1, _D))
    zg = _run_gatherz(z, pos_flat)
    out2d = _run_combine(x2d, zg, w01,
                         sw1, sb1.reshape(1, _FFS),
                         sw3, sb3.reshape(1, _FFS),
                         sw2, sb2.reshape(1, _D))
    return out2d.reshape(bsz, seq_len, d)
